# baseline (speedup 1.0000x reference)
import functools

import jax
import jax.numpy as jnp
from jax import lax
from jax.experimental import pallas as pl
from jax.experimental.pallas import tpu as pltpu
from jax.experimental.pallas import tpu_sc as plsc

_NC = 2
_NS = 16
_NW = _NC * _NS

_CH = 125
_KB = 32
_NBUF = 4


def _mlp1_body(x_ref, w1_ref, b1_ref, w2_ref, b2_ref, o_ref):
    h = jnp.dot(x_ref[...], w1_ref[...], preferred_element_type=jnp.float32)
    h = jnp.maximum(h + b1_ref[...], 0.0)
    o_ref[...] = (
        jnp.dot(h, w2_ref[...], preferred_element_type=jnp.float32) + b2_ref[...]
    )


def _mlp2_body(x_ref, sp_ref, c0_ref, c1_ref, w3a_ref, w3b_ref, b3_ref,
               w4_ref, b4_ref, w5_ref, b5_ref, o_ref):
    s = sp_ref[0] + sp_ref[1]
    cnt = c0_ref[...] + c1_ref[...]
    agg = s / jnp.maximum(cnt, 1.0)
    h = jnp.dot(x_ref[...], w3a_ref[...], preferred_element_type=jnp.float32)
    h = h + jnp.dot(agg, w3b_ref[...], preferred_element_type=jnp.float32)
    h = jnp.maximum(h + b3_ref[...], 0.0)
    h = jnp.maximum(
        jnp.dot(h, w4_ref[...], preferred_element_type=jnp.float32) + b4_ref[...],
        0.0,
    )
    o_ref[...] = (
        jnp.dot(h, w5_ref[...], preferred_element_type=jnp.float32) + b5_ref[...]
    )


def _sc_segment_sum(fx, ei3, z16, z1, n_acc, t_ch):
    nblk = t_ch // _KB
    rpt = n_acc // _NS
    half = rpt // 2

    mesh = plsc.VectorSubcoreMesh(core_axis_name="c", subcore_axis_name="s")

    @functools.partial(
        pl.kernel,
        out_type=(
            jax.ShapeDtypeStruct((_NC, n_acc, 16), jnp.float32),
            jax.ShapeDtypeStruct((n_acc,), jnp.float32),
            jax.ShapeDtypeStruct((n_acc,), jnp.float32),
        ),
        mesh=mesh,
        compiler_params=pltpu.CompilerParams(use_tc_tiling_on_sc=False),
        scratch_types=[
            pltpu.VMEM_SHARED((n_acc, 16), jnp.float32),
            pltpu.VMEM_SHARED((n_acc,), jnp.float32),
            pltpu.VMEM((_KB, _CH), jnp.int32),
            pltpu.VMEM((_KB, _CH), jnp.int32),
            pltpu.VMEM((_NBUF, _CH, 16), jnp.float32),
            pltpu.VMEM((128,), jnp.float32),
            pltpu.VMEM((half,), jnp.float32),
            pltpu.SemaphoreType.DMA((_NBUF,)),
        ],
    )
    def k(fx_hbm, ei_hbm, z16_hbm, z1_hbm, sums_hbm, cnt0_hbm, cnt1_hbm,
          acc, cnta, rowv, colv, gbuf, ones_v, cstage, gsem):
        c = lax.axis_index("c")
        s = lax.axis_index("s")
        wid = s * _NC + c
        base_chunk = wid * t_ch

        for i in range(8):
            ones_v[pl.ds(i * 16, 16)] = jnp.ones((16,), jnp.float32)

        pltpu.sync_copy(z16_hbm.at[pl.ds(s * rpt, rpt)],
                        acc.at[pl.ds(s * rpt, rpt)])
        for i in range(2):
            pltpu.sync_copy(z1_hbm.at[pl.ds(s * rpt + i * half, half)], cstage)
            pltpu.sync_copy(cstage, cnta.at[pl.ds(s * rpt + i * half, half)])
        plsc.subcore_barrier()

        def issue_gather(j):
            slot = lax.rem(j, _NBUF)
            return pltpu.async_copy(
                fx_hbm.at[rowv.at[j]], gbuf.at[slot], gsem.at[slot]
            )

        def wait_gather(j):
            slot = lax.rem(j, _NBUF)
            pltpu.make_async_copy(
                fx_hbm.at[rowv.at[j]], gbuf.at[slot], gsem.at[slot]
            ).wait()

        def block_body(b, carry):
            blk = base_chunk + b * _KB
            pltpu.sync_copy(ei_hbm.at[0, pl.ds(blk, _KB)], rowv)
            pltpu.sync_copy(ei_hbm.at[1, pl.ds(blk, _KB)], colv)
            for p in range(_NBUF - 1):
                issue_gather(p)

            def chunk_body(j, carry2):
                slot = lax.rem(j, _NBUF)
                wait_gather(j)
                pltpu.sync_copy(gbuf.at[slot], acc.at[colv.at[j]], add=True)
                pltpu.sync_copy(ones_v.at[pl.ds(0, _CH)],
                                cnta.at[colv.at[j]], add=True)

                @pl.when(j + _NBUF - 1 < _KB)
                def _():
                    issue_gather(j + _NBUF - 1)

                return carry2

            return lax.fori_loop(0, _KB, chunk_body, carry)

        lax.fori_loop(0, nblk, block_body, 0)
        plsc.subcore_barrier()

        pltpu.sync_copy(acc.at[pl.ds(s * rpt, rpt)],
                        sums_hbm.at[c, pl.ds(s * rpt, rpt)])
        for i in range(2):
            pltpu.sync_copy(cnta.at[pl.ds(s * rpt + i * half, half)], cstage)

            @pl.when(c == 0)
            def _():
                pltpu.sync_copy(cstage,
                                cnt0_hbm.at[pl.ds(s * rpt + i * half, half)])

            @pl.when(c == 1)
            def _():
                pltpu.sync_copy(cstage,
                                cnt1_hbm.at[pl.ds(s * rpt + i * half, half)])

    return k(fx, ei3, z16, z1)


def kernel(x, edge_index, edge_attr, u, batch, W1, b1, W2, b2, W3, b3,
           W4, b4, W5, b5):
    n = x.shape[0]
    e = edge_index.shape[1]

    rpt = (n + (8 * _NS) - 1) // (8 * _NS) * 8
    n_acc = rpt * _NS

    chunks = e // _CH
    t_ch = chunks // _NW
    ei3 = edge_index.reshape(2, chunks, _CH)

    b1r = b1.reshape(1, 16)
    b2r = b2.reshape(1, 16)
    b3r = b3.reshape(1, 16)
    b4r = b4.reshape(1, 16)
    b5r = b5.reshape(1, 3)
    W3a = W3[:3]
    W3b = W3[3:]

    x_pad = jnp.pad(x, ((0, n_acc - n), (0, 0)))

    rb = n_acc // 16
    grid = (16,)

    fx = pl.pallas_call(
        _mlp1_body,
        grid=grid,
        in_specs=[
            pl.BlockSpec((rb, 3), lambda i: (i, 0)),
            pl.BlockSpec((3, 16), lambda i: (0, 0)),
            pl.BlockSpec((1, 16), lambda i: (0, 0)),
            pl.BlockSpec((16, 16), lambda i: (0, 0)),
            pl.BlockSpec((1, 16), lambda i: (0, 0)),
        ],
        out_specs=pl.BlockSpec((rb, 16), lambda i: (i, 0)),
        out_shape=jax.ShapeDtypeStruct((n_acc, 16), jnp.float32),
    )(x_pad, W1, b1r, W2, b2r)

    z16 = jnp.zeros((n_acc, 16), jnp.float32)
    z1 = jnp.zeros((n_acc,), jnp.float32)
    sums_p, cnt0, cnt1 = _sc_segment_sum(fx, ei3, z16, z1, n_acc, t_ch)

    out = pl.pallas_call(
        _mlp2_body,
        grid=grid,
        in_specs=[
            pl.BlockSpec((rb, 3), lambda i: (i, 0)),
            pl.BlockSpec((2, rb, 16), lambda i: (0, i, 0)),
            pl.BlockSpec((rb, 1), lambda i: (i, 0)),
            pl.BlockSpec((rb, 1), lambda i: (i, 0)),
            pl.BlockSpec((3, 16), lambda i: (0, 0)),
            pl.BlockSpec((16, 16), lambda i: (0, 0)),
            pl.BlockSpec((1, 16), lambda i: (0, 0)),
            pl.BlockSpec((16, 16), lambda i: (0, 0)),
            pl.BlockSpec((1, 16), lambda i: (0, 0)),
            pl.BlockSpec((16, 3), lambda i: (0, 0)),
            pl.BlockSpec((1, 3), lambda i: (0, 0)),
        ],
        out_specs=pl.BlockSpec((rb, 3), lambda i: (i, 0)),
        out_shape=jax.ShapeDtypeStruct((n_acc, 3), jnp.float32),
    )(x_pad, sums_p, cnt0.reshape(n_acc, 1), cnt1.reshape(n_acc, 1),
      W3a, W3b, b3r, W4, b4r, W5, b5r)

    return out[:n]


# speedup vs baseline: 1.6956x; 1.6956x over previous
import functools

import jax
import jax.numpy as jnp
from jax import lax
from jax.experimental import pallas as pl
from jax.experimental.pallas import tpu as pltpu
from jax.experimental.pallas import tpu_sc as plsc

_NC = 2
_NS = 16
_NW = _NC * _NS

_CH = 128
_KB = 32
_NBUF = 4


def _mlp1_body(x_ref, w1_ref, b1_ref, w2_ref, b2_ref, o_ref):
    h = jnp.dot(x_ref[...], w1_ref[...], preferred_element_type=jnp.float32)
    h = jnp.maximum(h + b1_ref[...], 0.0)
    o_ref[...] = (
        jnp.dot(h, w2_ref[...], preferred_element_type=jnp.float32) + b2_ref[...]
    )


def _mlp2_body(x_ref, sp_ref, c0_ref, c1_ref, w3a_ref, w3b_ref, b3_ref,
               w4_ref, b4_ref, w5_ref, b5_ref, o_ref):
    s = sp_ref[0] + sp_ref[1]
    cnt = c0_ref[...] + c1_ref[...]
    agg = s / jnp.maximum(cnt, 1.0)
    h = jnp.dot(x_ref[...], w3a_ref[...], preferred_element_type=jnp.float32)
    h = h + jnp.dot(agg, w3b_ref[...], preferred_element_type=jnp.float32)
    h = jnp.maximum(h + b3_ref[...], 0.0)
    h = jnp.maximum(
        jnp.dot(h, w4_ref[...], preferred_element_type=jnp.float32) + b4_ref[...],
        0.0,
    )
    o_ref[...] = (
        jnp.dot(h, w5_ref[...], preferred_element_type=jnp.float32) + b5_ref[...]
    )


def _sc_segment_sum(fx, row2d, col2d, z16, z1, n_acc, t_ch):
    nblk = t_ch // _KB
    rpt = n_acc // _NS
    half = rpt // 2

    mesh = plsc.VectorSubcoreMesh(core_axis_name="c", subcore_axis_name="s")

    @functools.partial(
        pl.kernel,
        out_type=(
            jax.ShapeDtypeStruct((_NC, n_acc, 16), jnp.float32),
            jax.ShapeDtypeStruct((n_acc,), jnp.float32),
            jax.ShapeDtypeStruct((n_acc,), jnp.float32),
        ),
        mesh=mesh,
        compiler_params=pltpu.CompilerParams(use_tc_tiling_on_sc=False),
        scratch_types=[
            pltpu.VMEM_SHARED((n_acc, 16), jnp.float32),
            pltpu.VMEM_SHARED((n_acc,), jnp.float32),
            pltpu.VMEM((_KB, _CH), jnp.int32),
            pltpu.VMEM((_KB, _CH), jnp.int32),
            pltpu.VMEM((_NBUF, _CH, 16), jnp.float32),
            pltpu.VMEM((_CH,), jnp.float32),
            pltpu.VMEM((half,), jnp.float32),
            pltpu.SemaphoreType.DMA((_NBUF,)),
        ],
    )
    def k(fx_hbm, row_hbm, col_hbm, z16_hbm, z1_hbm, sums_hbm, cnt0_hbm,
          cnt1_hbm, acc, cnta, rowv, colv, gbuf, ones_v, cstage, gsem):
        c = lax.axis_index("c")
        s = lax.axis_index("s")
        wid = s * _NC + c
        base_chunk = wid * t_ch

        for i in range(8):
            ones_v[pl.ds(i * 16, 16)] = jnp.ones((16,), jnp.float32)

        pltpu.sync_copy(z16_hbm.at[pl.ds(s * rpt, rpt)],
                        acc.at[pl.ds(s * rpt, rpt)])
        for i in range(2):
            pltpu.sync_copy(z1_hbm.at[pl.ds(s * rpt + i * half, half)], cstage)
            pltpu.sync_copy(cstage, cnta.at[pl.ds(s * rpt + i * half, half)])
        plsc.subcore_barrier()

        def issue_gather(j):
            slot = lax.rem(j, _NBUF)
            return pltpu.async_copy(
                fx_hbm.at[rowv.at[j]], gbuf.at[slot], gsem.at[slot]
            )

        def wait_gather(j):
            slot = lax.rem(j, _NBUF)
            pltpu.make_async_copy(
                fx_hbm.at[rowv.at[j]], gbuf.at[slot], gsem.at[slot]
            ).wait()

        def block_body(b, carry):
            blk = base_chunk + b * _KB
            pltpu.sync_copy(row_hbm.at[pl.ds(blk, _KB)], rowv)
            pltpu.sync_copy(col_hbm.at[pl.ds(blk, _KB)], colv)
            for p in range(_NBUF - 1):
                issue_gather(p)

            def chunk_body(j, carry2):
                slot = lax.rem(j, _NBUF)
                wait_gather(j)
                pltpu.sync_copy(gbuf.at[slot], acc.at[colv.at[j]], add=True)
                pltpu.sync_copy(ones_v, cnta.at[colv.at[j]], add=True)

                @pl.when(j + _NBUF - 1 < _KB)
                def _():
                    issue_gather(j + _NBUF - 1)

                return carry2

            return lax.fori_loop(0, _KB, chunk_body, carry)

        lax.fori_loop(0, nblk, block_body, 0)
        plsc.subcore_barrier()

        pltpu.sync_copy(acc.at[pl.ds(s * rpt, rpt)],
                        sums_hbm.at[c, pl.ds(s * rpt, rpt)])
        for i in range(2):
            pltpu.sync_copy(cnta.at[pl.ds(s * rpt + i * half, half)], cstage)

            @pl.when(c == 0)
            def _():
                pltpu.sync_copy(cstage,
                                cnt0_hbm.at[pl.ds(s * rpt + i * half, half)])

            @pl.when(c == 1)
            def _():
                pltpu.sync_copy(cstage,
                                cnt1_hbm.at[pl.ds(s * rpt + i * half, half)])

    return k(fx, row2d, col2d, z16, z1)


def kernel(x, edge_index, edge_attr, u, batch, W1, b1, W2, b2, W3, b3,
           W4, b4, W5, b5):
    n = x.shape[0]
    e = edge_index.shape[1]

    rpt = (n + (8 * _NS) - 1) // (8 * _NS) * 8
    n_acc = rpt * _NS

    per_tile = (e + _NW * _CH - 1) // (_NW * _CH)
    t_ch = (per_tile + _KB - 1) // _KB * _KB
    e_pad = _NW * t_ch * _CH
    pad = e_pad - e
    row2d = jnp.concatenate(
        [edge_index[0], jnp.zeros((pad,), jnp.int32)]).reshape(-1, _CH)
    col2d = jnp.concatenate(
        [edge_index[1], jnp.full((pad,), n, jnp.int32)]).reshape(-1, _CH)

    b1r = b1.reshape(1, 16)
    b2r = b2.reshape(1, 16)
    b3r = b3.reshape(1, 16)
    b4r = b4.reshape(1, 16)
    b5r = b5.reshape(1, 3)
    W3a = W3[:3]
    W3b = W3[3:]

    x_pad = jnp.pad(x, ((0, n_acc - n), (0, 0)))

    rb = n_acc // 16
    grid = (16,)

    fx = pl.pallas_call(
        _mlp1_body,
        grid=grid,
        in_specs=[
            pl.BlockSpec((rb, 3), lambda i: (i, 0)),
            pl.BlockSpec((3, 16), lambda i: (0, 0)),
            pl.BlockSpec((1, 16), lambda i: (0, 0)),
            pl.BlockSpec((16, 16), lambda i: (0, 0)),
            pl.BlockSpec((1, 16), lambda i: (0, 0)),
        ],
        out_specs=pl.BlockSpec((rb, 16), lambda i: (i, 0)),
        out_shape=jax.ShapeDtypeStruct((n_acc, 16), jnp.float32),
    )(x_pad, W1, b1r, W2, b2r)

    z16 = jnp.zeros((n_acc, 16), jnp.float32)
    z1 = jnp.zeros((n_acc,), jnp.float32)
    sums_p, cnt0, cnt1 = _sc_segment_sum(fx, row2d, col2d, z16, z1,
                                         n_acc, t_ch)

    out = pl.pallas_call(
        _mlp2_body,
        grid=grid,
        in_specs=[
            pl.BlockSpec((rb, 3), lambda i: (i, 0)),
            pl.BlockSpec((2, rb, 16), lambda i: (0, i, 0)),
            pl.BlockSpec((rb, 1), lambda i: (i, 0)),
            pl.BlockSpec((rb, 1), lambda i: (i, 0)),
            pl.BlockSpec((3, 16), lambda i: (0, 0)),
            pl.BlockSpec((16, 16), lambda i: (0, 0)),
            pl.BlockSpec((1, 16), lambda i: (0, 0)),
            pl.BlockSpec((16, 16), lambda i: (0, 0)),
            pl.BlockSpec((1, 16), lambda i: (0, 0)),
            pl.BlockSpec((16, 3), lambda i: (0, 0)),
            pl.BlockSpec((1, 3), lambda i: (0, 0)),
        ],
        out_specs=pl.BlockSpec((rb, 3), lambda i: (i, 0)),
        out_shape=jax.ShapeDtypeStruct((n_acc, 3), jnp.float32),
    )(x_pad, sums_p, cnt0.reshape(n_acc, 1), cnt1.reshape(n_acc, 1),
      W3a, W3b, b3r, W4, b4r, W5, b5r)

    return out[:n]


# speedup vs baseline: 1.7611x; 1.0386x over previous
import functools

import jax
import jax.numpy as jnp
from jax import lax
from jax.experimental import pallas as pl
from jax.experimental.pallas import tpu as pltpu
from jax.experimental.pallas import tpu_sc as plsc

_NC = 2
_NS = 16
_NW = _NC * _NS

_CH = 128
_KB = 32
_NBUF = 4


def _mlp1_body(x_ref, w1_ref, b1_ref, w2_ref, b2_ref, o_ref):
    h = jnp.dot(x_ref[...], w1_ref[...], preferred_element_type=jnp.float32)
    h = jnp.maximum(h + b1_ref[...], 0.0)
    o_ref[...] = (
        jnp.dot(h, w2_ref[...], preferred_element_type=jnp.float32) + b2_ref[...]
    )


def _mlp2_body(x_ref, sp_ref, c0_ref, c1_ref, w3a_ref, w3b_ref, b3_ref,
               w4_ref, b4_ref, w5_ref, b5_ref, o_ref):
    s = sp_ref[0] + sp_ref[1]
    cnt = c0_ref[...] + c1_ref[...]
    agg = s / jnp.maximum(cnt, 1.0)
    h = jnp.dot(x_ref[...], w3a_ref[...], preferred_element_type=jnp.float32)
    h = h + jnp.dot(agg, w3b_ref[...], preferred_element_type=jnp.float32)
    h = jnp.maximum(h + b3_ref[...], 0.0)
    h = jnp.maximum(
        jnp.dot(h, w4_ref[...], preferred_element_type=jnp.float32) + b4_ref[...],
        0.0,
    )
    o_ref[...] = (
        jnp.dot(h, w5_ref[...], preferred_element_type=jnp.float32) + b5_ref[...]
    )


def _sc_segment_sum(fx, row2d, col2d, z16, z1, n_acc, t_ch):
    nblk = t_ch // _KB
    rpt = n_acc // _NS
    half = rpt // 2

    mesh = plsc.VectorSubcoreMesh(core_axis_name="c", subcore_axis_name="s")

    @functools.partial(
        pl.kernel,
        out_type=(
            jax.ShapeDtypeStruct((_NC, n_acc, 16), jnp.float32),
            jax.ShapeDtypeStruct((n_acc,), jnp.float32),
            jax.ShapeDtypeStruct((n_acc,), jnp.float32),
        ),
        mesh=mesh,
        compiler_params=pltpu.CompilerParams(use_tc_tiling_on_sc=False),
        scratch_types=[
            pltpu.VMEM_SHARED((n_acc, 16), jnp.float32),
            pltpu.VMEM_SHARED((n_acc,), jnp.float32),
            pltpu.VMEM((_KB, _CH), jnp.int32),
            pltpu.VMEM((_KB, _CH), jnp.int32),
            pltpu.VMEM((_NBUF, _CH, 16), jnp.float32),
            pltpu.VMEM((_CH,), jnp.float32),
            pltpu.VMEM((half,), jnp.float32),
            pltpu.SemaphoreType.DMA((_NBUF,)),
            pltpu.SemaphoreType.DMA((_NBUF,)),
        ],
    )
    def k(fx_hbm, row_hbm, col_hbm, z16_hbm, z1_hbm, sums_hbm, cnt0_hbm,
          cnt1_hbm, acc, cnta, rowv, colv, gbuf, ones_v, cstage, gsem, ssem):
        c = lax.axis_index("c")
        s = lax.axis_index("s")
        wid = s * _NC + c
        base_chunk = wid * t_ch

        for i in range(8):
            ones_v[pl.ds(i * 16, 16)] = jnp.ones((16,), jnp.float32)

        pltpu.sync_copy(z16_hbm.at[pl.ds(s * rpt, rpt)],
                        acc.at[pl.ds(s * rpt, rpt)])
        for i in range(2):
            pltpu.sync_copy(z1_hbm.at[pl.ds(s * rpt + i * half, half)], cstage)
            pltpu.sync_copy(cstage, cnta.at[pl.ds(s * rpt + i * half, half)])
        plsc.subcore_barrier()

        def issue_gather(j):
            slot = lax.rem(j, _NBUF)
            return pltpu.async_copy(
                fx_hbm.at[rowv.at[j]], gbuf.at[slot], gsem.at[slot]
            )

        def wait_gather(j):
            slot = lax.rem(j, _NBUF)
            pltpu.make_async_copy(
                fx_hbm.at[rowv.at[j]], gbuf.at[slot], gsem.at[slot]
            ).wait()

        def issue_scatter(j):
            slot = lax.rem(j, _NBUF)
            return pltpu.async_copy(
                gbuf.at[slot], acc.at[colv.at[j]], ssem.at[slot], add=True
            )

        def wait_scatter(j):
            slot = lax.rem(j, _NBUF)
            pltpu.make_async_copy(
                gbuf.at[slot], acc.at[colv.at[j]], ssem.at[slot]
            ).wait()

        def block_body(b, carry):
            @pl.when(b > 0)
            def _():
                wait_scatter(_KB - 1)

            blk = base_chunk + b * _KB
            pltpu.sync_copy(row_hbm.at[pl.ds(blk, _KB)], rowv)
            pltpu.sync_copy(col_hbm.at[pl.ds(blk, _KB)], colv)
            for p in range(_NBUF - 1):
                issue_gather(p)

            def chunk_body(j, carry2):
                wait_gather(j)
                issue_scatter(j)
                pltpu.sync_copy(ones_v, cnta.at[colv.at[j]], add=True)

                @pl.when(j > 0)
                def _():
                    wait_scatter(j - 1)

                @pl.when(j + _NBUF - 1 < _KB)
                def _():
                    issue_gather(j + _NBUF - 1)

                return carry2

            return lax.fori_loop(0, _KB, chunk_body, carry)

        lax.fori_loop(0, nblk, block_body, 0)
        wait_scatter(_KB - 1)
        plsc.subcore_barrier()

        pltpu.sync_copy(acc.at[pl.ds(s * rpt, rpt)],
                        sums_hbm.at[c, pl.ds(s * rpt, rpt)])
        for i in range(2):
            pltpu.sync_copy(cnta.at[pl.ds(s * rpt + i * half, half)], cstage)

            @pl.when(c == 0)
            def _():
                pltpu.sync_copy(cstage,
                                cnt0_hbm.at[pl.ds(s * rpt + i * half, half)])

            @pl.when(c == 1)
            def _():
                pltpu.sync_copy(cstage,
                                cnt1_hbm.at[pl.ds(s * rpt + i * half, half)])

    return k(fx, row2d, col2d, z16, z1)


def kernel(x, edge_index, edge_attr, u, batch, W1, b1, W2, b2, W3, b3,
           W4, b4, W5, b5):
    n = x.shape[0]
    e = edge_index.shape[1]

    rpt = (n + (8 * _NS) - 1) // (8 * _NS) * 8
    n_acc = rpt * _NS

    per_tile = (e + _NW * _CH - 1) // (_NW * _CH)
    t_ch = (per_tile + _KB - 1) // _KB * _KB
    e_pad = _NW * t_ch * _CH
    pad = e_pad - e
    row2d = jnp.concatenate(
        [edge_index[0], jnp.zeros((pad,), jnp.int32)]).reshape(-1, _CH)
    col2d = jnp.concatenate(
        [edge_index[1], jnp.full((pad,), n, jnp.int32)]).reshape(-1, _CH)

    b1r = b1.reshape(1, 16)
    b2r = b2.reshape(1, 16)
    b3r = b3.reshape(1, 16)
    b4r = b4.reshape(1, 16)
    b5r = b5.reshape(1, 3)
    W3a = W3[:3]
    W3b = W3[3:]

    x_pad = jnp.pad(x, ((0, n_acc - n), (0, 0)))

    rb = n_acc // 16
    grid = (16,)

    fx = pl.pallas_call(
        _mlp1_body,
        grid=grid,
        in_specs=[
            pl.BlockSpec((rb, 3), lambda i: (i, 0)),
            pl.BlockSpec((3, 16), lambda i: (0, 0)),
            pl.BlockSpec((1, 16), lambda i: (0, 0)),
            pl.BlockSpec((16, 16), lambda i: (0, 0)),
            pl.BlockSpec((1, 16), lambda i: (0, 0)),
        ],
        out_specs=pl.BlockSpec((rb, 16), lambda i: (i, 0)),
        out_shape=jax.ShapeDtypeStruct((n_acc, 16), jnp.float32),
    )(x_pad, W1, b1r, W2, b2r)

    z16 = jnp.zeros((n_acc, 16), jnp.float32)
    z1 = jnp.zeros((n_acc,), jnp.float32)
    sums_p, cnt0, cnt1 = _sc_segment_sum(fx, row2d, col2d, z16, z1,
                                         n_acc, t_ch)

    out = pl.pallas_call(
        _mlp2_body,
        grid=grid,
        in_specs=[
            pl.BlockSpec((rb, 3), lambda i: (i, 0)),
            pl.BlockSpec((2, rb, 16), lambda i: (0, i, 0)),
            pl.BlockSpec((rb, 1), lambda i: (i, 0)),
            pl.BlockSpec((rb, 1), lambda i: (i, 0)),
            pl.BlockSpec((3, 16), lambda i: (0, 0)),
            pl.BlockSpec((16, 16), lambda i: (0, 0)),
            pl.BlockSpec((1, 16), lambda i: (0, 0)),
            pl.BlockSpec((16, 16), lambda i: (0, 0)),
            pl.BlockSpec((1, 16), lambda i: (0, 0)),
            pl.BlockSpec((16, 3), lambda i: (0, 0)),
            pl.BlockSpec((1, 3), lambda i: (0, 0)),
        ],
        out_specs=pl.BlockSpec((rb, 3), lambda i: (i, 0)),
        out_shape=jax.ShapeDtypeStruct((n_acc, 3), jnp.float32),
    )(x_pad, sums_p, cnt0.reshape(n_acc, 1), cnt1.reshape(n_acc, 1),
      W3a, W3b, b3r, W4, b4r, W5, b5r)

    return out[:n]
